# baseline (device time: 22117 ns/iter reference)
import jax
import jax.numpy as jnp
from jax import lax
from jax.experimental import pallas as pl
from jax.experimental.pallas import tpu as pltpu

B, H, D = 8, 8, 64
P_LOCAL = 64
BS = 16
NSLOTS = 64
T = P_LOCAL * BS
CW = 128
NEG = -1e30


def _body(q_ref, k_ref, v_ref, bt_ref, lens_ref,
          out_ref, send_ref, recv_ref, send_sem, recv_sem):
    my_x = lax.axis_index("x")
    my_y = lax.axis_index("y")
    my_z = lax.axis_index("z")
    peer = (my_x, my_y, 1 - my_z)

    barrier = pltpu.get_barrier_semaphore()
    pl.semaphore_signal(barrier, inc=1, device_id=peer,
                        device_id_type=pl.DeviceIdType.MESH)
    pl.semaphore_wait(barrier, 1)

    lens_col = jnp.stack([lens_ref[i] for i in range(B)]).reshape(B, 1)
    slot = lax.broadcasted_iota(jnp.int32, (B, NSLOTS), 1)
    btv = jnp.where(slot < lens_col, bt_ref[:, :], -1)
    pid = my_z * P_LOCAL + lax.broadcasted_iota(jnp.int32, (1, 1, P_LOCAL), 2)
    counts = jnp.sum((btv[:, :, None] == pid).astype(jnp.int32), axis=1)
    w = jnp.broadcast_to(counts[:, :, None], (B, P_LOCAL, BS))
    w = w.reshape(B, T).astype(jnp.float32)

    q = q_ref[:, 0, :, :].astype(jnp.bfloat16)
    kt = k_ref[...].reshape(T, H, D).astype(jnp.bfloat16)
    vt = v_ref[...].reshape(T, H, D).astype(jnp.bfloat16)
    qh = jnp.transpose(q, (1, 0, 2))
    kh = jnp.transpose(kt, (1, 2, 0))
    vh = jnp.transpose(vt, (1, 0, 2))

    s = lax.dot_general(qh, kh, (((2,), (1,)), ((0,), (0,))),
                        preferred_element_type=jnp.float32)
    s = s * (D ** -0.5)
    wb = w[None, :, :]
    s = jnp.where(wb > 0, s, NEG)
    m = jnp.max(s, axis=2)
    p = jnp.exp(s - m[:, :, None]) * wb
    l = jnp.sum(p, axis=2)
    o = lax.dot_general(p.astype(jnp.bfloat16), vh,
                        (((2,), (1,)), ((0,), (0,))),
                        preferred_element_type=jnp.float32)

    pad = jnp.zeros((H, B, CW - D - 2), jnp.float32)
    send_ref[...] = jnp.concatenate(
        [o, m[:, :, None], l[:, :, None], pad], axis=2)

    rdma = pltpu.make_async_remote_copy(
        src_ref=send_ref, dst_ref=recv_ref,
        send_sem=send_sem, recv_sem=recv_sem,
        device_id=peer, device_id_type=pl.DeviceIdType.MESH,
    )
    rdma.start()
    rdma.wait()

    rcv = recv_ref[...]
    o2, m2, l2 = rcv[:, :, 0:D], rcv[:, :, D], rcv[:, :, D + 1]

    mn = jnp.maximum(m, m2)
    a = jnp.exp(m - mn)
    b = jnp.exp(m2 - mn)
    ln = l * a + l2 * b
    on = (o * a[:, :, None] + o2 * b[:, :, None]) / ln[:, :, None]

    out_ref[...] = jnp.transpose(on, (1, 0, 2)).reshape(B, 1, H, D)


def kernel(Q, K, V, bt, lens):
    return pl.pallas_call(
        _body,
        out_shape=jax.ShapeDtypeStruct((B, 1, H, D), jnp.float32),
        in_specs=[
            pl.BlockSpec(memory_space=pltpu.VMEM),
            pl.BlockSpec(memory_space=pltpu.VMEM),
            pl.BlockSpec(memory_space=pltpu.VMEM),
            pl.BlockSpec(memory_space=pltpu.VMEM),
            pl.BlockSpec(memory_space=pltpu.SMEM),
        ],
        out_specs=pl.BlockSpec(memory_space=pltpu.VMEM),
        scratch_shapes=[
            pltpu.VMEM((H, B, CW), jnp.float32),
            pltpu.VMEM((H, B, CW), jnp.float32),
            pltpu.SemaphoreType.DMA,
            pltpu.SemaphoreType.DMA,
        ],
        compiler_params=pltpu.CompilerParams(collective_id=0),
    )(Q, K, V, bt, lens)


# device time: 14901 ns/iter; 1.4843x vs baseline; 1.4843x over previous
import jax
import jax.numpy as jnp
from jax import lax
from jax.experimental import pallas as pl
from jax.experimental.pallas import tpu as pltpu

B, H, D = 8, 8, 64
P_LOCAL = 64
BS = 16
NSLOTS = 64
T = P_LOCAL * BS
HB = H * B
CROWS = 72
NEG = -1e30


def _body(q_ref, k_ref, v_ref, bt_ref, lens_ref,
          out_ref, send_ref, recv_ref, send_sem, recv_sem):
    my_x = lax.axis_index("x")
    my_y = lax.axis_index("y")
    my_z = lax.axis_index("z")
    peer = (my_x, my_y, 1 - my_z)

    barrier = pltpu.get_barrier_semaphore()
    pl.semaphore_signal(barrier, inc=1, device_id=peer,
                        device_id_type=pl.DeviceIdType.MESH)
    pl.semaphore_wait(barrier, 1)

    lens_col = jnp.stack([lens_ref[i] for i in range(B)]).reshape(B, 1)
    slot = lax.broadcasted_iota(jnp.int32, (B, NSLOTS), 1)
    btv = jnp.where(slot < lens_col, bt_ref[:, :], -1)
    pid_t = my_z * P_LOCAL + lax.broadcasted_iota(
        jnp.int32, (P_LOCAL, 1, 1), 0)
    counts_t = jnp.sum((pid_t == btv[None, :, :]).astype(jnp.int32), axis=2)
    w3 = jnp.broadcast_to(counts_t[:, None, :], (P_LOCAL, BS, B)).reshape(T, B)
    w_hb = jnp.tile(w3, (1, H)).astype(jnp.float32)

    q = q_ref[:, 0, :, :].astype(jnp.bfloat16)
    qp = jnp.transpose(q, (1, 2, 0)).reshape(H * D, B)
    qbd_full = jnp.tile(qp, (1, H))
    row_h = lax.broadcasted_iota(jnp.int32, (H * D, HB), 0) // D
    col_h = lax.broadcasted_iota(jnp.int32, (H * D, HB), 1) // B
    qbd = jnp.where(row_h == col_h, qbd_full, jnp.bfloat16(0))

    kflat = k_ref[...].reshape(T, H * D).astype(jnp.bfloat16)
    vflat = v_ref[...].reshape(T, H * D).astype(jnp.bfloat16)

    s_t = lax.dot_general(kflat, qbd, (((1,), (0,)), ((), ())),
                          preferred_element_type=jnp.float32)
    s_t = s_t * (D ** -0.5)
    s_t = jnp.where(w_hb > 0, s_t, NEG)
    m_t = jnp.max(s_t, axis=0, keepdims=True)
    p_t = (jnp.exp(s_t - m_t) * w_hb).astype(jnp.bfloat16)
    l_t = jnp.sum(p_t.astype(jnp.float32), axis=0, keepdims=True)

    o2t = lax.dot_general(vflat, p_t, (((0,), (0,)), ((), ())),
                          preferred_element_type=jnp.float32)
    o_pack = jnp.concatenate(
        [o2t[h * D:(h + 1) * D, h * B:(h + 1) * B] for h in range(H)],
        axis=1)

    send_ref[...] = jnp.concatenate(
        [o_pack, m_t, l_t, jnp.zeros((CROWS - D - 2, HB), jnp.float32)],
        axis=0)

    rdma = pltpu.make_async_remote_copy(
        src_ref=send_ref, dst_ref=recv_ref,
        send_sem=send_sem, recv_sem=recv_sem,
        device_id=peer, device_id_type=pl.DeviceIdType.MESH,
    )
    rdma.start()
    rdma.wait()

    rcv = recv_ref[...]
    o2r = rcv[0:D, :]
    m2 = rcv[D:D + 1, :]
    l2 = rcv[D + 1:D + 2, :]

    mn = jnp.maximum(m_t, m2)
    a = jnp.exp(m_t - mn)
    b = jnp.exp(m2 - mn)
    ln = l_t * a + l2 * b
    on = (o_pack * a + o2r * b) / ln
    on_t = jnp.transpose(on)
    for h in range(H):
        out_ref[:, 0, h, :] = on_t[h * B:(h + 1) * B, :]


def kernel(Q, K, V, bt, lens):
    return pl.pallas_call(
        _body,
        out_shape=jax.ShapeDtypeStruct((B, 1, H, D), jnp.float32),
        in_specs=[
            pl.BlockSpec(memory_space=pltpu.VMEM),
            pl.BlockSpec(memory_space=pltpu.VMEM),
            pl.BlockSpec(memory_space=pltpu.VMEM),
            pl.BlockSpec(memory_space=pltpu.VMEM),
            pl.BlockSpec(memory_space=pltpu.SMEM),
        ],
        out_specs=pl.BlockSpec(memory_space=pltpu.VMEM),
        scratch_shapes=[
            pltpu.VMEM((CROWS, HB), jnp.float32),
            pltpu.VMEM((CROWS, HB), jnp.float32),
            pltpu.SemaphoreType.DMA,
            pltpu.SemaphoreType.DMA,
        ],
        compiler_params=pltpu.CompilerParams(collective_id=0),
    )(Q, K, V, bt, lens)
